# baseline (speedup 1.0000x reference)
import functools

import jax
import jax.numpy as jnp
from jax import lax
from jax.experimental import pallas as pl
from jax.experimental.pallas import tpu as pltpu
from jax.experimental.pallas import tpu_sc as plsc

NC = 2
NS = 16
L = 16
NW = NC * NS

D = 128
DV = D // L
S = 200
B = 1024
BPW = B // NW
TPW = BPW * S
EPS = 1e-12
TUNROLL = 5
NBUF = 3
C1 = 104
C2 = S - C1

_mesh = plsc.VectorSubcoreMesh(core_axis_name="c", subcore_axis_name="s")


@functools.partial(
    pl.kernel,
    out_type=jax.ShapeDtypeStruct((B * S, D), jnp.float32),
    mesh=_mesh,
    scratch_types=[
        pltpu.VMEM((TPW,), jnp.int32),
        pltpu.VMEM((S, D), jnp.float32),
        pltpu.VMEM((S, D), jnp.float32),
        pltpu.VMEM((S, D), jnp.float32),
        pltpu.VMEM((S, D), jnp.float32),
        pltpu.VMEM((D,), jnp.float32),
        pltpu.VMEM((D,), jnp.float32),
        pltpu.SemaphoreType.DMA,
        pltpu.SemaphoreType.DMA,
        pltpu.SemaphoreType.DMA,
        pltpu.SemaphoreType.DMA,
        pltpu.SemaphoreType.DMA,
        pltpu.SemaphoreType.DMA,
    ],
    compiler_params=pltpu.CompilerParams(needs_layout_passes=False),
)
def _bert_embed(x_hbm, ww_hbm, pos_hbm, g_hbm, b_hbm, out_hbm,
                idx_v, rows0, rows1, rows2, pos_v, g_v, b_v,
                in0, in1, in2, out0, out1, out2):
    rows = [rows0, rows1, rows2]
    isems = [in0, in1, in2]
    osems = [out0, out1, out2]

    wid = lax.axis_index("s") * NC + lax.axis_index("c")
    tok0 = wid * TPW

    pltpu.sync_copy(pos_hbm, pos_v)
    pltpu.sync_copy(g_hbm, g_v)
    pltpu.sync_copy(b_hbm, b_v)
    pltpu.sync_copy(x_hbm.at[pl.ds(tok0, TPW)], idx_v)

    gregs = [g_v[pl.ds(k * L, L)] for k in range(DV)]
    bregs = [b_v[pl.ds(k * L, L)] for k in range(DV)]

    def start_gather(tbase, buf, sem):
        pltpu.async_copy(ww_hbm.at[idx_v.at[pl.ds(tbase, C1)]],
                         buf.at[pl.ds(0, C1)], sem)
        pltpu.async_copy(ww_hbm.at[idx_v.at[pl.ds(tbase + C1, C2)]],
                         buf.at[pl.ds(C1, C2)], sem)

    def wait_gather(buf, sem):
        pltpu.make_async_copy(ww_hbm.at[idx_v.at[pl.ds(0, C1)]],
                              buf.at[pl.ds(0, C1)], sem).wait()
        pltpu.make_async_copy(ww_hbm.at[idx_v.at[pl.ds(0, C2)]],
                              buf.at[pl.ds(C1, C2)], sem).wait()

    def wait_out(buf, sem):
        pltpu.make_async_copy(buf, out_hbm.at[pl.ds(0, S)], sem).wait()

    def compute_batch(buf):
        def tok_body(ti, carry2):
          for u in range(TUNROLL):
            t = ti * TUNROLL + u
            e = [buf[t, pl.ds(k * L, L)] + pos_v[t, pl.ds(k * L, L)]
                 for k in range(DV)]
            v = list(e)
            while len(v) > 1:
                v = [v[m] + v[m + 1] for m in range(0, len(v), 2)]
            s1 = v[0]
            w = [ek * ek for ek in e]
            while len(w) > 1:
                w = [w[m] + w[m + 1] for m in range(0, len(w), 2)]
            s2 = w[0]
            tot1 = lax.reduce_sum_p.bind(s1, axes=(0,))
            tot2 = lax.reduce_sum_p.bind(s2, axes=(0,))
            mean = tot1 * (1.0 / D)
            var = tot2 * (1.0 / D) - mean * mean
            a = var + EPS
            ibits = lax.bitcast_convert_type(a, jnp.int32)
            ibits = jnp.int32(0x5F3759DF) - lax.shift_right_logical(ibits, 1)
            y = lax.bitcast_convert_type(ibits, jnp.float32)
            half = a * 0.5
            for _ in range(2):
                y = y * (1.5 - half * (y * y))
            istd = y
            for k in range(DV):
                o = (e[k] - mean) * (istd * gregs[k]) + bregs[k]
                buf[t, pl.ds(k * L, L)] = o
          return carry2

        lax.fori_loop(0, S // TUNROLL, tok_body, 0)

    start_gather(0, rows[0], isems[0])

    def step_body(step, carry):
        for j in range(NBUF):
            i = step * NBUF + j
            q = (j + 1) % NBUF
            cb = lax.rem(i, BPW) * S
            nb = lax.rem(i + 1, BPW) * S

            @pl.when(i >= 2)
            def _():
                wait_out(rows[q], osems[q])
            start_gather(nb, rows[q], isems[q])
            wait_gather(rows[j], isems[j])
            compute_batch(rows[j])
            pltpu.async_copy(rows[j], out_hbm.at[pl.ds(tok0 + cb, S)],
                             osems[j])
        return carry

    lax.fori_loop(0, (BPW + 1) // NBUF, step_body, 0)

    wait_out(rows[1], osems[1])
    wait_out(rows[2], osems[2])
    wait_gather(rows[0], isems[0])


def kernel(x, W_word, W_pos, gamma, beta):
    x_flat = x.reshape(-1).astype(jnp.int32)
    out = _bert_embed(x_flat, W_word, W_pos[:S].astype(jnp.float32),
                      gamma, beta)
    return out.reshape(B, S, D)


# speedup vs baseline: 1.0228x; 1.0228x over previous
import functools

import jax
import jax.numpy as jnp
from jax import lax
from jax.experimental import pallas as pl
from jax.experimental.pallas import tpu as pltpu
from jax.experimental.pallas import tpu_sc as plsc

NC = 2
NS = 16
L = 16
NW = NC * NS

D = 128
DV = D // L
S = 200
B = 1024
BPW = B // NW
TPW = BPW * S
EPS = 1e-12
TUNROLL = 5
NBUF = 3
C1 = 104
C2 = S - C1

_mesh = plsc.VectorSubcoreMesh(core_axis_name="c", subcore_axis_name="s")


@functools.partial(
    pl.kernel,
    out_type=jax.ShapeDtypeStruct((B * S, D), jnp.float32),
    mesh=_mesh,
    scratch_types=[
        pltpu.VMEM((TPW,), jnp.int32),
        pltpu.VMEM((S, D), jnp.float32),
        pltpu.VMEM((S, D), jnp.float32),
        pltpu.VMEM((S, D), jnp.float32),
        pltpu.VMEM((S, D), jnp.float32),
        pltpu.VMEM((D,), jnp.float32),
        pltpu.VMEM((D,), jnp.float32),
        pltpu.SemaphoreType.DMA,
        pltpu.SemaphoreType.DMA,
        pltpu.SemaphoreType.DMA,
        pltpu.SemaphoreType.DMA,
        pltpu.SemaphoreType.DMA,
        pltpu.SemaphoreType.DMA,
    ],
    compiler_params=pltpu.CompilerParams(needs_layout_passes=False),
)
def _bert_embed(x_hbm, ww_hbm, pos_hbm, g_hbm, b_hbm, out_hbm,
                idx_v, rows0, rows1, rows2, pos_v, g_v, b_v,
                in0, in1, in2, out0, out1, out2):
    rows = [rows0, rows1, rows2]
    isems = [in0, in1, in2]
    osems = [out0, out1, out2]

    wid = lax.axis_index("s") * NC + lax.axis_index("c")
    tok0 = wid * TPW

    pltpu.sync_copy(pos_hbm, pos_v)
    pltpu.sync_copy(g_hbm, g_v)
    pltpu.sync_copy(b_hbm, b_v)
    pltpu.sync_copy(x_hbm.at[pl.ds(tok0, TPW)], idx_v)

    gregs = [g_v[pl.ds(k * L, L)] for k in range(DV)]
    bregs = [b_v[pl.ds(k * L, L)] for k in range(DV)]

    def start_gather(tbase, buf, sem):
        pltpu.async_copy(ww_hbm.at[idx_v.at[pl.ds(tbase, C1)]],
                         buf.at[pl.ds(0, C1)], sem)
        pltpu.async_copy(ww_hbm.at[idx_v.at[pl.ds(tbase + C1, C2)]],
                         buf.at[pl.ds(C1, C2)], sem)

    def wait_gather(buf, sem):
        pltpu.make_async_copy(ww_hbm.at[idx_v.at[pl.ds(0, C1)]],
                              buf.at[pl.ds(0, C1)], sem).wait()
        pltpu.make_async_copy(ww_hbm.at[idx_v.at[pl.ds(0, C2)]],
                              buf.at[pl.ds(C1, C2)], sem).wait()

    def wait_out(buf, sem):
        pltpu.make_async_copy(buf, out_hbm.at[pl.ds(0, S)], sem).wait()

    def compute_batch(buf):
        def tok_body(ti, carry2):
          for u in range(TUNROLL):
            t = ti * TUNROLL + u
            e = [buf[t, pl.ds(k * L, L)] + pos_v[t, pl.ds(k * L, L)]
                 for k in range(DV)]
            s1 = e[0]
            for k in range(1, DV):
                s1 = s1 + e[k]
            s2 = e[0] * e[0]
            for k in range(1, DV):
                s2 = s2 + e[k] * e[k]
            tot1 = lax.reduce_sum_p.bind(s1, axes=(0,))
            tot2 = lax.reduce_sum_p.bind(s2, axes=(0,))
            mean = tot1 * (1.0 / D)
            var = tot2 * (1.0 / D) - mean * mean
            a = var + EPS
            ibits = lax.bitcast_convert_type(a, jnp.int32)
            ibits = jnp.int32(0x5F3759DF) - lax.shift_right_logical(ibits, 1)
            y = lax.bitcast_convert_type(ibits, jnp.float32)
            half = a * 0.5
            for _ in range(2):
                y = y * (1.5 - half * (y * y))
            istd = y
            for k in range(DV):
                o = (e[k] - mean) * (istd * gregs[k]) + bregs[k]
                buf[t, pl.ds(k * L, L)] = o
          return carry2

        lax.fori_loop(0, S // TUNROLL, tok_body, 0)

    start_gather(0, rows[0], isems[0])

    def step_body(step, carry):
        for j in range(NBUF):
            i = step * NBUF + j
            q = (j + 1) % NBUF
            cb = lax.rem(i, BPW) * S
            nb = lax.rem(i + 1, BPW) * S

            @pl.when(i >= 2)
            def _():
                wait_out(rows[q], osems[q])
            start_gather(nb, rows[q], isems[q])
            wait_gather(rows[j], isems[j])
            compute_batch(rows[j])
            pltpu.async_copy(rows[j], out_hbm.at[pl.ds(tok0 + cb, S)],
                             osems[j])
        return carry

    lax.fori_loop(0, (BPW + 1) // NBUF, step_body, 0)

    wait_out(rows[1], osems[1])
    wait_out(rows[2], osems[2])
    wait_gather(rows[0], isems[0])


def kernel(x, W_word, W_pos, gamma, beta):
    x_flat = x.reshape(-1).astype(jnp.int32)
    out = _bert_embed(x_flat, W_word, W_pos[:S].astype(jnp.float32),
                      gamma, beta)
    return out.reshape(B, S, D)
